# baseline (speedup 1.0000x reference)
import dataclasses
import functools

import jax
import jax.numpy as jnp
from jax import lax
from jax.experimental import pallas as pl
from jax.experimental.pallas import tpu as pltpu
from jax.experimental.pallas import tpu_sc as plsc

_B, _S, _H = 16, 4096, 128
_EPS = 1e-05
_HALF = _S // 2
_K = 128
_NSUB = _HALF // _K
_L = 16


def _sc_embed_sum(state_flat, statep_flat, token_table, instruction_table,
                  argument_table):
    mesh = plsc.VectorSubcoreMesh(core_axis_name="c", subcore_axis_name="s")
    cp = pltpu.CompilerParams()
    if "needs_layout_passes" in pltpu.CompilerParams.__dataclass_fields__:
        cp = dataclasses.replace(cp, needs_layout_passes=False)

    @functools.partial(
        pl.kernel,
        mesh=mesh,
        compiler_params=cp,
        out_type=jax.ShapeDtypeStruct((_B * _S, _H), jnp.float32),
        scratch_types=[
            pltpu.VMEM((_HALF,), jnp.int32),
            pltpu.VMEM((_HALF,), jnp.int32),
            pltpu.VMEM((_HALF,), jnp.int32),
            pltpu.VMEM((_HALF,), jnp.int32),
            pltpu.VMEM((_K, _H), jnp.float32),
            pltpu.VMEM((_K, _H), jnp.float32),
            pltpu.VMEM((_K, _H), jnp.float32),
            pltpu.SemaphoreType.DMA,
            pltpu.SemaphoreType.DMA,
            pltpu.SemaphoreType.DMA,
        ],
    )
    def body(state_hbm, statep_hbm, tok_hbm, ins_hbm, arg_hbm, out_hbm,
             sv, spv, iiv, aiv, tokb, insb, argb, sem0, sem1, sem2):
        row = lax.axis_index("s")
        half = lax.axis_index("c")
        row_base = row * _S
        base = row_base + half * _HALF

        iota = lax.iota(jnp.int32, 16)

        def _prescan(_):
            pltpu.sync_copy(statep_hbm.at[pl.ds(row_base, _HALF)], spv)

            def pre_body(g, carry):
                ci, cs = carry
                sp16 = spv[pl.ds(g * _L, _L)]
                starts = sp16 == 0
                ci = ci + jnp.sum(jnp.where(starts, 1, 0).astype(jnp.int32))
                pos16 = iota + g * _L
                cs = jnp.maximum(
                    cs, jnp.max(jnp.where(starts, pos16, 0)))
                return ci, cs

            return lax.fori_loop(0, _HALF // _L, pre_body,
                                 (jnp.int32(0), jnp.int32(0)))

        ci0, cs0 = lax.cond(half == 1, _prescan,
                            lambda _: (jnp.int32(0), jnp.int32(0)), 0)

        pltpu.sync_copy(state_hbm.at[pl.ds(base, _HALF)], sv)
        pltpu.sync_copy(statep_hbm.at[pl.ds(base, _HALF)], spv)

        local0 = half * _HALF

        def idx_body(j, carry):
            ci, cs = carry
            for g in range(8):
                off = j * _K + g * _L
                sp16 = spv[pl.ds(off, _L)]
                starts = sp16 == 0
                s16 = jnp.where(starts, 1, 0).astype(jnp.int32)
                inst16 = plsc.cumsum(s16) + ci
                ci = jnp.max(inst16)
                pos16 = iota + (local0 + off)
                mpos = jnp.where(starts, pos16, 0)
                seg16 = jnp.maximum(plsc.cummax(mpos), cs)
                cs = jnp.max(seg16)
                iiv[pl.ds(off, _L)] = inst16
                aiv[pl.ds(off, _L)] = pos16 - seg16
            return ci, cs

        lax.fori_loop(0, _NSUB, idx_body, (ci0, cs0))

        def chunk_body(j, _):
            sl = pl.ds(j * _K, _K)
            cp0 = pltpu.async_copy(tok_hbm.at[sv.at[sl]], tokb, sem0)
            cp1 = pltpu.async_copy(ins_hbm.at[iiv.at[sl]], insb, sem1)
            cp2 = pltpu.async_copy(arg_hbm.at[aiv.at[sl]], argb, sem2)
            cp0.wait()
            cp1.wait()
            cp2.wait()

            def add_body(t, c):
                for hh in range(_H // _L):
                    s2 = pl.ds(hh * _L, _L)
                    tokb[t, s2] = tokb[t, s2] + insb[t, s2] + argb[t, s2]
                return c

            lax.fori_loop(0, _K, add_body, 0)
            pltpu.sync_copy(tokb, out_hbm.at[pl.ds(base + j * _K, _K)])
            return 0

        lax.fori_loop(0, _NSUB, chunk_body, 0)

    return body(state_flat, statep_flat, token_table, instruction_table,
                argument_table)


def _ln_body(x_ref, w_ref, b_ref, o_ref):
    x = x_ref[...]
    mu = jnp.mean(x, axis=-1, keepdims=True)
    xc = x - mu
    var = jnp.mean(xc * xc, axis=-1, keepdims=True)
    inv = lax.rsqrt(var + _EPS)
    o_ref[...] = xc * inv * w_ref[...] + b_ref[...]


def _layernorm(summed, ln_weight, ln_bias):
    blk = 2048
    return pl.pallas_call(
        _ln_body,
        grid=(_B * _S // blk,),
        in_specs=[
            pl.BlockSpec((blk, _H), lambda i: (i, 0)),
            pl.BlockSpec((1, _H), lambda i: (0, 0)),
            pl.BlockSpec((1, _H), lambda i: (0, 0)),
        ],
        out_specs=pl.BlockSpec((blk, _H), lambda i: (i, 0)),
        out_shape=jax.ShapeDtypeStruct((_B * _S, _H), jnp.float32),
    )(summed, ln_weight.reshape(1, _H), ln_bias.reshape(1, _H))


def kernel(state, token_table, instruction_table, argument_table, ln_weight,
           ln_bias):
    state = state.astype(jnp.int32)
    statep = jnp.roll(state, 1, axis=-1).at[:, 0].set(1)
    summed = _sc_embed_sum(state.reshape(-1), statep.reshape(-1),
                           token_table, instruction_table, argument_table)
    out = _layernorm(summed, ln_weight, ln_bias)
    return out.reshape(_B, _S, _H)


# speedup vs baseline: 12.5878x; 12.5878x over previous
import dataclasses
import functools

import jax
import jax.numpy as jnp
from jax import lax
from jax.experimental import pallas as pl
from jax.experimental.pallas import tpu as pltpu
from jax.experimental.pallas import tpu_sc as plsc

_B, _S, _H = 16, 4096, 128
_EPS = 1e-05
_HALF = _S // 2
_K = 128
_NSUB = _HALF // _K
_L = 16


def _sc_embed_sum(state_flat, statep_flat, token_table, instruction_table,
                  argument_table):
    mesh = plsc.VectorSubcoreMesh(core_axis_name="c", subcore_axis_name="s")
    cp = pltpu.CompilerParams()
    if "needs_layout_passes" in pltpu.CompilerParams.__dataclass_fields__:
        cp = dataclasses.replace(cp, needs_layout_passes=False)

    @functools.partial(
        pl.kernel,
        mesh=mesh,
        compiler_params=cp,
        out_type=jax.ShapeDtypeStruct((_B * _S, _H), jnp.float32),
        scratch_types=[
            pltpu.VMEM((_HALF,), jnp.int32),
            pltpu.VMEM((_HALF,), jnp.int32),
            pltpu.VMEM((_HALF,), jnp.int32),
            pltpu.VMEM((_HALF,), jnp.int32),
            pltpu.VMEM((_K, _H), jnp.float32),
            pltpu.VMEM((_K, _H), jnp.float32),
            pltpu.VMEM((_K + 8, _H), jnp.float32),
            pltpu.SMEM((_NSUB,), jnp.int32),
            pltpu.SMEM((_NSUB,), jnp.int32),
            pltpu.SMEM((_NSUB,), jnp.int32),
            pltpu.SemaphoreType.DMA,
            pltpu.SemaphoreType.DMA,
            pltpu.SemaphoreType.DMA,
        ],
    )
    def body(state_hbm, statep_hbm, tok_hbm, ins_hbm, arg_hbm, out_hbm,
             sv, spv, iiv, aiv, tokb, insb, argb, ns_sm, ci_sm, ba_sm,
             sem0, sem1, sem2):
        row = lax.axis_index("s")
        half = lax.axis_index("c")
        row_base = row * _S
        base = row_base + half * _HALF

        iota = lax.iota(jnp.int32, 16)

        def _prescan(_):
            pltpu.sync_copy(statep_hbm.at[pl.ds(row_base, _HALF)], spv)

            def pre_body(g, carry):
                ci, cs = carry
                sp16 = spv[pl.ds(g * _L, _L)]
                starts = sp16 == 0
                ci = ci + jnp.sum(jnp.where(starts, 1, 0).astype(jnp.int32))
                pos16 = iota + g * _L
                cs = jnp.maximum(
                    cs, jnp.max(jnp.where(starts, pos16, 0)))
                return ci, cs

            return lax.fori_loop(0, _HALF // _L, pre_body,
                                 (jnp.int32(0), jnp.int32(0)))

        ci0, cs0 = lax.cond(half == 1, _prescan,
                            lambda _: (jnp.int32(0), jnp.int32(0)), 0)

        pltpu.sync_copy(state_hbm.at[pl.ds(base, _HALF)], sv)
        pltpu.sync_copy(statep_hbm.at[pl.ds(base, _HALF)], spv)

        local0 = half * _HALF

        def idx_body(j, carry):
            ci, cs = carry
            ci_sm[j] = ci
            ba_sm[j] = local0 + j * _K - cs
            ci_in = ci
            for g in range(8):
                off = j * _K + g * _L
                sp16 = spv[pl.ds(off, _L)]
                starts = sp16 == 0
                s16 = jnp.where(starts, 1, 0).astype(jnp.int32)
                inst16 = plsc.cumsum(s16) + ci
                ci = jnp.max(inst16)
                pos16 = iota + (local0 + off)
                mpos = jnp.where(starts, pos16, 0)
                seg16 = jnp.maximum(plsc.cummax(mpos), cs)
                cs = jnp.max(seg16)
                iiv[pl.ds(off, _L)] = inst16
                aiv[pl.ds(off, _L)] = pos16 - seg16
            ns_sm[j] = ci - ci_in
            return ci, cs

        lax.fori_loop(0, _NSUB, idx_body, (ci0, cs0))

        def chunk_body(j, _):
            sl = pl.ds(j * _K, _K)
            cp0 = pltpu.async_copy(tok_hbm.at[sv.at[sl]], tokb, sem0)
            ns = ns_sm[j]
            ci_j = ci_sm[j]
            ci8 = pl.multiple_of(
                jnp.minimum(ci_j - lax.rem(ci_j, 8), 4096 - 8), 8)
            ba_j = ba_sm[j]
            ba8 = pl.multiple_of(
                jnp.minimum(ba_j - lax.rem(ba_j, 8), 4096 - (_K + 8)), 8)

            def fast(_):
                cpi = pltpu.async_copy(ins_hbm.at[pl.ds(ci8, 8)],
                                       insb.at[pl.ds(0, 8)], sem1)
                cpa = pltpu.async_copy(arg_hbm.at[pl.ds(ba8, _K + 8)],
                                       argb, sem2)
                cpi.wait()
                cpa.wait()
                return 0

            def slow(_):
                cpi = pltpu.async_copy(ins_hbm.at[iiv.at[sl]], insb, sem1)
                cpa = pltpu.async_copy(arg_hbm.at[aiv.at[sl]],
                                       argb.at[pl.ds(0, _K)], sem2)
                cpi.wait()
                cpa.wait()
                return 0

            lax.cond(ns == 0, fast, slow, 0)
            cp0.wait()
            io = ci_j - ci8
            ao = ba_j - ba8

            def add_body(t, c):
                r = jnp.where(ns == 0, io, t)
                a = jnp.where(ns == 0, t + ao, t)
                for hh in range(_H // _L):
                    s2 = pl.ds(hh * _L, _L)
                    tokb[t, s2] = tokb[t, s2] + insb[r, s2] + argb[a, s2]
                return c

            lax.fori_loop(0, _K, add_body, 0)
            pltpu.sync_copy(tokb, out_hbm.at[pl.ds(base + j * _K, _K)])
            return 0

        lax.fori_loop(0, _NSUB, chunk_body, 0)

    return body(state_flat, statep_flat, token_table, instruction_table,
                argument_table)


def _ln_body(x_ref, w_ref, b_ref, o_ref):
    x = x_ref[...]
    mu = jnp.mean(x, axis=-1, keepdims=True)
    xc = x - mu
    var = jnp.mean(xc * xc, axis=-1, keepdims=True)
    inv = lax.rsqrt(var + _EPS)
    o_ref[...] = xc * inv * w_ref[...] + b_ref[...]


def _layernorm(summed, ln_weight, ln_bias):
    blk = 2048
    return pl.pallas_call(
        _ln_body,
        grid=(_B * _S // blk,),
        in_specs=[
            pl.BlockSpec((blk, _H), lambda i: (i, 0)),
            pl.BlockSpec((1, _H), lambda i: (0, 0)),
            pl.BlockSpec((1, _H), lambda i: (0, 0)),
        ],
        out_specs=pl.BlockSpec((blk, _H), lambda i: (i, 0)),
        out_shape=jax.ShapeDtypeStruct((_B * _S, _H), jnp.float32),
    )(summed, ln_weight.reshape(1, _H), ln_bias.reshape(1, _H))


def kernel(state, token_table, instruction_table, argument_table, ln_weight,
           ln_bias):
    state = state.astype(jnp.int32)
    statep = jnp.roll(state, 1, axis=-1).at[:, 0].set(1)
    summed = _sc_embed_sum(state.reshape(-1), statep.reshape(-1),
                           token_table, instruction_table, argument_table)
    out = _layernorm(summed, ln_weight, ln_bias)
    return out.reshape(_B, _S, _H)


# speedup vs baseline: 14.6349x; 1.1626x over previous
import dataclasses
import functools

import jax
import jax.numpy as jnp
from jax import lax
from jax.experimental import pallas as pl
from jax.experimental.pallas import tpu as pltpu
from jax.experimental.pallas import tpu_sc as plsc

_B, _S, _H = 16, 4096, 128
_EPS = 1e-05
_HALF = _S // 2
_K = 128
_NSUB = _HALF // _K
_L = 16


def _sc_embed_sum(state_flat, statep_flat, token_table, instruction_table,
                  argument_table):
    mesh = plsc.VectorSubcoreMesh(core_axis_name="c", subcore_axis_name="s")
    cp = pltpu.CompilerParams()
    if "needs_layout_passes" in pltpu.CompilerParams.__dataclass_fields__:
        cp = dataclasses.replace(cp, needs_layout_passes=False)

    @functools.partial(
        pl.kernel,
        mesh=mesh,
        compiler_params=cp,
        out_type=jax.ShapeDtypeStruct((_B * _S, _H), jnp.float32),
        scratch_types=[
            pltpu.VMEM((_HALF,), jnp.int32),
            pltpu.VMEM((_HALF,), jnp.int32),
            pltpu.VMEM((_HALF,), jnp.int32),
            pltpu.VMEM((_HALF,), jnp.int32),
            pltpu.VMEM((2, _K, _H), jnp.float32),
            pltpu.VMEM((2, _K, _H), jnp.float32),
            pltpu.VMEM((2, _K + 8, _H), jnp.float32),
            pltpu.SMEM((_NSUB,), jnp.int32),
            pltpu.SMEM((_NSUB,), jnp.int32),
            pltpu.SMEM((_NSUB,), jnp.int32),
            pltpu.SemaphoreType.DMA,
            pltpu.SemaphoreType.DMA,
            pltpu.SemaphoreType.DMA,
            pltpu.SemaphoreType.DMA,
            pltpu.SemaphoreType.DMA,
            pltpu.SemaphoreType.DMA,
            pltpu.SemaphoreType.DMA,
            pltpu.SemaphoreType.DMA,
        ],
    )
    def body(state_hbm, statep_hbm, tok_hbm, ins_hbm, arg_hbm, out_hbm,
             sv, spv, iiv, aiv, tokb2, insb2, argb2, ns_sm, ci_sm, ba_sm,
             st0, st1, si0, si1, sa0, sa1, so0, so1):
        row = lax.axis_index("s")
        half = lax.axis_index("c")
        row_base = row * _S
        base = row_base + half * _HALF

        iota = lax.iota(jnp.int32, 16)

        def _prescan(_):
            pltpu.sync_copy(statep_hbm.at[pl.ds(row_base, _HALF)], spv)

            def pre_body(g, carry):
                ci, cs = carry
                sp16 = spv[pl.ds(g * _L, _L)]
                starts = sp16 == 0
                ci = ci + jnp.sum(jnp.where(starts, 1, 0).astype(jnp.int32))
                pos16 = iota + g * _L
                cs = jnp.maximum(
                    cs, jnp.max(jnp.where(starts, pos16, 0)))
                return ci, cs

            return lax.fori_loop(0, _HALF // _L, pre_body,
                                 (jnp.int32(0), jnp.int32(0)))

        ci0, cs0 = lax.cond(half == 1, _prescan,
                            lambda _: (jnp.int32(0), jnp.int32(0)), 0)

        pltpu.sync_copy(state_hbm.at[pl.ds(base, _HALF)], sv)
        pltpu.sync_copy(statep_hbm.at[pl.ds(base, _HALF)], spv)

        local0 = half * _HALF

        def idx_body(j, carry):
            ci, cs = carry
            ci_sm[j] = ci
            ba_sm[j] = local0 + j * _K - cs
            ci_in = ci
            for g in range(8):
                off = j * _K + g * _L
                sp16 = spv[pl.ds(off, _L)]
                starts = sp16 == 0
                s16 = jnp.where(starts, 1, 0).astype(jnp.int32)
                inst16 = plsc.cumsum(s16) + ci
                ci = jnp.max(inst16)
                pos16 = iota + (local0 + off)
                mpos = jnp.where(starts, pos16, 0)
                seg16 = jnp.maximum(plsc.cummax(mpos), cs)
                cs = jnp.max(seg16)
                iiv[pl.ds(off, _L)] = inst16
                aiv[pl.ds(off, _L)] = pos16 - seg16
            ns_sm[j] = ci - ci_in
            return ci, cs

        lax.fori_loop(0, _NSUB, idx_body, (ci0, cs0))

        sem_tok = (st0, st1)
        sem_ins = (si0, si1)
        sem_arg = (sa0, sa1)
        sem_out = (so0, so1)

        def _bases(j):
            ci_j = ci_sm[j]
            ci8 = pl.multiple_of(
                jnp.minimum(ci_j - lax.rem(ci_j, 8), 4096 - 8), 8)
            ba_j = ba_sm[j]
            ba8 = pl.multiple_of(
                jnp.minimum(ba_j - lax.rem(ba_j, 8), 4096 - (_K + 8)), 8)
            return ci_j, ci8, ba_j, ba8

        def issue(j, p):
            sl = pl.ds(j * _K, _K)
            pltpu.async_copy(tok_hbm.at[sv.at[sl]], tokb2.at[p], sem_tok[p])
            ns = ns_sm[j]
            _, ci8, _, ba8 = _bases(j)

            def fast(_):
                pltpu.async_copy(ins_hbm.at[pl.ds(ci8, 8)],
                                 insb2.at[p].at[pl.ds(0, 8)], sem_ins[p])
                pltpu.async_copy(arg_hbm.at[pl.ds(ba8, _K + 8)],
                                 argb2.at[p], sem_arg[p])
                return 0

            def slow(_):
                pltpu.async_copy(ins_hbm.at[iiv.at[sl]],
                                 insb2.at[p].at[pl.ds(0, _K)], sem_ins[p])
                pltpu.async_copy(arg_hbm.at[aiv.at[sl]],
                                 argb2.at[p].at[pl.ds(0, _K)], sem_arg[p])
                return 0

            lax.cond(ns == 0, fast, slow, 0)

        def consume(j, p):
            sl = pl.ds(j * _K, _K)
            ns = ns_sm[j]
            ci_j, ci8, ba_j, ba8 = _bases(j)

            def dfast(_):
                pltpu.make_async_copy(ins_hbm.at[pl.ds(0, 8)],
                                      insb2.at[p].at[pl.ds(0, 8)],
                                      sem_ins[p]).wait()
                pltpu.make_async_copy(arg_hbm.at[pl.ds(0, _K + 8)],
                                      argb2.at[p], sem_arg[p]).wait()
                return 0

            def dslow(_):
                pltpu.make_async_copy(ins_hbm.at[pl.ds(0, _K)],
                                      insb2.at[p].at[pl.ds(0, _K)],
                                      sem_ins[p]).wait()
                pltpu.make_async_copy(arg_hbm.at[pl.ds(0, _K)],
                                      argb2.at[p].at[pl.ds(0, _K)],
                                      sem_arg[p]).wait()
                return 0

            lax.cond(ns == 0, dfast, dslow, 0)
            pltpu.make_async_copy(tok_hbm.at[sv.at[sl]], tokb2.at[p],
                                  sem_tok[p]).wait()
            io = ci_j - ci8
            ao = ba_j - ba8

            def add_body(t, c):
                r = jnp.where(ns == 0, io, t)
                a = jnp.where(ns == 0, t + ao, t)
                for hh in range(_H // _L):
                    s2 = pl.ds(hh * _L, _L)
                    tokb2[p, t, s2] = (tokb2[p, t, s2] + insb2[p, r, s2]
                                       + argb2[p, a, s2])
                return c

            lax.fori_loop(0, _K, add_body, 0)
            pltpu.async_copy(tokb2.at[p],
                             out_hbm.at[pl.ds(base + j * _K, _K)], sem_out[p])

        def wait_out(j, p):
            pltpu.make_async_copy(tokb2.at[p],
                                  out_hbm.at[pl.ds(base + j * _K, _K)],
                                  sem_out[p]).wait()

        issue(0, 0)
        for j in range(_NSUB):
            p = j & 1
            if j + 1 < _NSUB:
                if j >= 1:
                    wait_out(j - 1, 1 - p)
                issue(j + 1, 1 - p)
            consume(j, p)
        wait_out(_NSUB - 2, 0)
        wait_out(_NSUB - 1, 1)

    return body(state_flat, statep_flat, token_table, instruction_table,
                argument_table)


def _ln_body(x_ref, w_ref, b_ref, o_ref):
    x = x_ref[...]
    mu = jnp.mean(x, axis=-1, keepdims=True)
    xc = x - mu
    var = jnp.mean(xc * xc, axis=-1, keepdims=True)
    inv = lax.rsqrt(var + _EPS)
    o_ref[...] = xc * inv * w_ref[...] + b_ref[...]


def _layernorm(summed, ln_weight, ln_bias):
    blk = 2048
    return pl.pallas_call(
        _ln_body,
        grid=(_B * _S // blk,),
        in_specs=[
            pl.BlockSpec((blk, _H), lambda i: (i, 0)),
            pl.BlockSpec((1, _H), lambda i: (0, 0)),
            pl.BlockSpec((1, _H), lambda i: (0, 0)),
        ],
        out_specs=pl.BlockSpec((blk, _H), lambda i: (i, 0)),
        out_shape=jax.ShapeDtypeStruct((_B * _S, _H), jnp.float32),
    )(summed, ln_weight.reshape(1, _H), ln_bias.reshape(1, _H))


def kernel(state, token_table, instruction_table, argument_table, ln_weight,
           ln_bias):
    state = state.astype(jnp.int32)
    statep = jnp.roll(state, 1, axis=-1).at[:, 0].set(1)
    summed = _sc_embed_sum(state.reshape(-1), statep.reshape(-1),
                           token_table, instruction_table, argument_table)
    out = _layernorm(summed, ln_weight, ln_bias)
    return out.reshape(_B, _S, _H)


# speedup vs baseline: 15.3224x; 1.0470x over previous
import dataclasses
import functools

import jax
import jax.numpy as jnp
from jax import lax
from jax.experimental import pallas as pl
from jax.experimental.pallas import tpu as pltpu
from jax.experimental.pallas import tpu_sc as plsc

_B, _S, _H = 16, 4096, 128
_EPS = 1e-05
_HALF = _S // 2
_K = 128
_NSUB = _HALF // _K
_L = 16


def _sc_embed_sum(state_flat, statep_flat, token_table, instruction_table,
                  argument_table):
    mesh = plsc.VectorSubcoreMesh(core_axis_name="c", subcore_axis_name="s")
    cp = pltpu.CompilerParams()
    if "needs_layout_passes" in pltpu.CompilerParams.__dataclass_fields__:
        cp = dataclasses.replace(cp, needs_layout_passes=False)

    @functools.partial(
        pl.kernel,
        mesh=mesh,
        compiler_params=cp,
        out_type=jax.ShapeDtypeStruct((_B * _S, _H), jnp.float32),
        scratch_types=[
            pltpu.VMEM((_HALF,), jnp.int32),
            pltpu.VMEM((_HALF,), jnp.int32),
            pltpu.VMEM((_HALF,), jnp.int32),
            pltpu.VMEM((_HALF,), jnp.int32),
            pltpu.VMEM((2, _K, _H), jnp.float32),
            pltpu.VMEM((2, _K, _H), jnp.float32),
            pltpu.VMEM((2, _K + 8, _H), jnp.float32),
            pltpu.SMEM((_NSUB,), jnp.int32),
            pltpu.SMEM((_NSUB,), jnp.int32),
            pltpu.SMEM((_NSUB,), jnp.int32),
            pltpu.SemaphoreType.DMA,
            pltpu.SemaphoreType.DMA,
            pltpu.SemaphoreType.DMA,
            pltpu.SemaphoreType.DMA,
            pltpu.SemaphoreType.DMA,
            pltpu.SemaphoreType.DMA,
            pltpu.SemaphoreType.DMA,
            pltpu.SemaphoreType.DMA,
        ],
    )
    def body(state_hbm, statep_hbm, tok_hbm, ins_hbm, arg_hbm, out_hbm,
             sv, spv, iiv, aiv, tokb2, insb2, argb2, ns_sm, ci_sm, ba_sm,
             st0, st1, si0, si1, sa0, sa1, so0, so1):
        row = lax.axis_index("s")
        half = lax.axis_index("c")
        row_base = row * _S
        base = row_base + half * _HALF

        iota = lax.iota(jnp.int32, 16)

        def _prescan(_):
            pltpu.sync_copy(statep_hbm.at[pl.ds(row_base, _HALF)], spv)

            def pre_body(g, carry):
                ci, cs = carry
                sp16 = spv[pl.ds(g * _L, _L)]
                starts = sp16 == 0
                ci = ci + jnp.sum(jnp.where(starts, 1, 0).astype(jnp.int32))
                pos16 = iota + g * _L
                cs = jnp.maximum(
                    cs, jnp.max(jnp.where(starts, pos16, 0)))
                return ci, cs

            return lax.fori_loop(0, _HALF // _L, pre_body,
                                 (jnp.int32(0), jnp.int32(0)))

        ci0, cs0 = lax.cond(half == 1, _prescan,
                            lambda _: (jnp.int32(0), jnp.int32(0)), 0)

        pltpu.sync_copy(state_hbm.at[pl.ds(base, _HALF)], sv)
        pltpu.sync_copy(statep_hbm.at[pl.ds(base, _HALF)], spv)

        local0 = half * _HALF

        def idx_body(j, carry):
            ci, cs = carry
            ci_sm[j] = ci
            ba_sm[j] = local0 + j * _K - cs
            ci_in = ci
            for g in range(8):
                off = j * _K + g * _L
                sp16 = spv[pl.ds(off, _L)]
                starts = sp16 == 0
                s16 = jnp.where(starts, 1, 0).astype(jnp.int32)
                inst16 = plsc.cumsum(s16) + ci
                ci = jnp.max(inst16)
                pos16 = iota + (local0 + off)
                mpos = jnp.where(starts, pos16, 0)
                seg16 = jnp.maximum(plsc.cummax(mpos), cs)
                cs = jnp.max(seg16)
                iiv[pl.ds(off, _L)] = inst16
                aiv[pl.ds(off, _L)] = pos16 - seg16
            ns_sm[j] = ci - ci_in
            return ci, cs

        lax.fori_loop(0, _NSUB, idx_body, (ci0, cs0))

        sem_tok = (st0, st1)
        sem_ins = (si0, si1)
        sem_arg = (sa0, sa1)
        sem_out = (so0, so1)

        def _bases(j):
            ci_j = ci_sm[j]
            ci8 = pl.multiple_of(
                jnp.minimum(ci_j - lax.rem(ci_j, 8), 4096 - 8), 8)
            ba_j = ba_sm[j]
            ba8 = pl.multiple_of(
                jnp.minimum(ba_j - lax.rem(ba_j, 8), 4096 - (_K + 8)), 8)
            return ci_j, ci8, ba_j, ba8

        def issue(j, p):
            sl = pl.ds(j * _K, _K)
            pltpu.async_copy(tok_hbm.at[sv.at[sl]], tokb2.at[p], sem_tok[p])
            ns = ns_sm[j]
            _, ci8, _, ba8 = _bases(j)

            def fast(_):
                pltpu.async_copy(ins_hbm.at[pl.ds(ci8, 8)],
                                 insb2.at[p].at[pl.ds(0, 8)], sem_ins[p])
                pltpu.async_copy(arg_hbm.at[pl.ds(ba8, _K + 8)],
                                 argb2.at[p], sem_arg[p])
                return 0

            def slow(_):
                pltpu.async_copy(ins_hbm.at[iiv.at[sl]],
                                 insb2.at[p].at[pl.ds(0, _K)], sem_ins[p])
                pltpu.async_copy(arg_hbm.at[aiv.at[sl]],
                                 argb2.at[p].at[pl.ds(0, _K)], sem_arg[p])
                return 0

            lax.cond(ns == 0, fast, slow, 0)

        def consume(j, p):
            sl = pl.ds(j * _K, _K)
            ns = ns_sm[j]
            ci_j, ci8, ba_j, ba8 = _bases(j)

            io = ci_j - ci8
            ao = ba_j - ba8

            def dfast(_):
                pltpu.make_async_copy(ins_hbm.at[pl.ds(0, 8)],
                                      insb2.at[p].at[pl.ds(0, 8)],
                                      sem_ins[p]).wait()
                pltpu.make_async_copy(arg_hbm.at[pl.ds(0, _K + 8)],
                                      argb2.at[p], sem_arg[p]).wait()
                pltpu.make_async_copy(tok_hbm.at[sv.at[sl]], tokb2.at[p],
                                      sem_tok[p]).wait()
                ivals = [insb2[p, io, pl.ds(hh * _L, _L)]
                         for hh in range(_H // _L)]

                def add_fast(t, c):
                    for hh in range(_H // _L):
                        s2 = pl.ds(hh * _L, _L)
                        tokb2[p, t, s2] = (tokb2[p, t, s2] + ivals[hh]
                                           + argb2[p, t + ao, s2])
                    return c

                lax.fori_loop(0, _K, add_fast, 0)
                return 0

            def dslow(_):
                pltpu.make_async_copy(ins_hbm.at[pl.ds(0, _K)],
                                      insb2.at[p].at[pl.ds(0, _K)],
                                      sem_ins[p]).wait()
                pltpu.make_async_copy(arg_hbm.at[pl.ds(0, _K)],
                                      argb2.at[p].at[pl.ds(0, _K)],
                                      sem_arg[p]).wait()
                pltpu.make_async_copy(tok_hbm.at[sv.at[sl]], tokb2.at[p],
                                      sem_tok[p]).wait()

                def add_body(t, c):
                    for hh in range(_H // _L):
                        s2 = pl.ds(hh * _L, _L)
                        tokb2[p, t, s2] = (tokb2[p, t, s2] + insb2[p, t, s2]
                                           + argb2[p, t, s2])
                    return c

                lax.fori_loop(0, _K, add_body, 0)
                return 0

            lax.cond(ns == 0, dfast, dslow, 0)
            pltpu.async_copy(tokb2.at[p],
                             out_hbm.at[pl.ds(base + j * _K, _K)], sem_out[p])

        def wait_out(j, p):
            pltpu.make_async_copy(tokb2.at[p],
                                  out_hbm.at[pl.ds(base + j * _K, _K)],
                                  sem_out[p]).wait()

        issue(0, 0)
        for j in range(_NSUB):
            p = j & 1
            if j + 1 < _NSUB:
                if j >= 1:
                    wait_out(j - 1, 1 - p)
                issue(j + 1, 1 - p)
            consume(j, p)
        wait_out(_NSUB - 2, 0)
        wait_out(_NSUB - 1, 1)

    return body(state_flat, statep_flat, token_table, instruction_table,
                argument_table)


def _ln_body(x_ref, w_ref, b_ref, o_ref):
    x = x_ref[...]
    mu = jnp.mean(x, axis=-1, keepdims=True)
    xc = x - mu
    var = jnp.mean(xc * xc, axis=-1, keepdims=True)
    inv = lax.rsqrt(var + _EPS)
    o_ref[...] = xc * inv * w_ref[...] + b_ref[...]


def _layernorm(summed, ln_weight, ln_bias):
    blk = 2048
    return pl.pallas_call(
        _ln_body,
        grid=(_B * _S // blk,),
        in_specs=[
            pl.BlockSpec((blk, _H), lambda i: (i, 0)),
            pl.BlockSpec((1, _H), lambda i: (0, 0)),
            pl.BlockSpec((1, _H), lambda i: (0, 0)),
        ],
        out_specs=pl.BlockSpec((blk, _H), lambda i: (i, 0)),
        out_shape=jax.ShapeDtypeStruct((_B * _S, _H), jnp.float32),
    )(summed, ln_weight.reshape(1, _H), ln_bias.reshape(1, _H))


def kernel(state, token_table, instruction_table, argument_table, ln_weight,
           ln_bias):
    state = state.astype(jnp.int32)
    statep = jnp.roll(state, 1, axis=-1).at[:, 0].set(1)
    summed = _sc_embed_sum(state.reshape(-1), statep.reshape(-1),
                           token_table, instruction_table, argument_table)
    out = _layernorm(summed, ln_weight, ln_bias)
    return out.reshape(_B, _S, _H)
